# baseline (speedup 1.0000x reference)
import jax
import jax.numpy as jnp
from jax.experimental import pallas as pl
from jax.experimental.pallas import tpu as pltpu

_NBUF = 4
_BM = 200


def _hbs_kernel(x_ref, w_ref, nb_hbm, o_ref, m_ref, nb_buf, sem):
    n = x_ref.shape[0]
    nblocks = n // _BM

    def copy_in(block, slot):
        return pltpu.make_async_copy(
            nb_hbm.at[pl.ds(block * _BM, _BM), :],
            nb_buf.at[slot],
            sem.at[slot],
        )

    for j in range(_NBUF):
        copy_in(j, j).start()

    m_ref[...] = jnp.dot(
        x_ref[...], w_ref[...], preferred_element_type=jnp.float32
    ).astype(jnp.bfloat16)

    for i in range(nblocks):
        slot = i % _NBUF
        copy_in(i, slot).wait()
        acc = jnp.dot(nb_buf[slot].astype(jnp.bfloat16), m_ref[...],
                      preferred_element_type=jnp.float32)
        o_ref[pl.ds(i * _BM, _BM), :] = jnp.maximum(acc, 0.0)
        nxt = i + _NBUF
        if nxt < nblocks:
            copy_in(nxt, slot).start()


def kernel(x_source, neighborhood, weight, weight2, weight3):
    n, d_in = x_source.shape
    d_out = weight.shape[1]

    out = pl.pallas_call(
        _hbs_kernel,
        in_specs=[
            pl.BlockSpec((n, d_in), lambda: (0, 0)),
            pl.BlockSpec((d_in, d_out), lambda: (0, 0)),
            pl.BlockSpec(memory_space=pl.ANY),
        ],
        out_specs=pl.BlockSpec((n, d_out), lambda: (0, 0)),
        out_shape=jax.ShapeDtypeStruct((n, d_out), jnp.float32),
        scratch_shapes=[
            pltpu.VMEM((n, d_out), jnp.bfloat16),
            pltpu.VMEM((_NBUF, _BM, n), jnp.float32),
            pltpu.SemaphoreType.DMA((_NBUF,)),
        ],
    )(x_source, weight, neighborhood)
    return out


# speedup vs baseline: 1.0191x; 1.0191x over previous
import jax
import jax.numpy as jnp
from jax.experimental import pallas as pl
from jax.experimental.pallas import tpu as pltpu


def _fused_kernel(x_ref, w_ref, nb_ref, o_ref, m_ref):
    @pl.when(pl.program_id(0) == 0)
    def _():
        m_ref[...] = jnp.dot(
            x_ref[...], w_ref[...], preferred_element_type=jnp.float32
        )

    acc = jnp.dot(nb_ref[...], m_ref[...],
                  preferred_element_type=jnp.float32)
    o_ref[...] = jnp.maximum(acc, 0.0)


def kernel(x_source, neighborhood, weight, weight2, weight3):
    n, d_in = x_source.shape
    d_out = weight.shape[1]

    bm = 400
    out = pl.pallas_call(
        _fused_kernel,
        grid=(n // bm,),
        in_specs=[
            pl.BlockSpec((n, d_in), lambda i: (0, 0)),
            pl.BlockSpec((d_in, d_out), lambda i: (0, 0)),
            pl.BlockSpec((bm, n), lambda i: (i, 0)),
        ],
        out_specs=pl.BlockSpec((bm, d_out), lambda i: (i, 0)),
        out_shape=jax.ShapeDtypeStruct((n, d_out), jnp.float32),
        scratch_shapes=[pltpu.VMEM((n, d_out), jnp.float32)],
        compiler_params=pltpu.CompilerParams(
            dimension_semantics=("arbitrary",),
        ),
    )(x_source, weight, neighborhood)
    return out
